# baseline (speedup 1.0000x reference)
import dataclasses
import functools

import jax
import jax.numpy as jnp
from jax import lax
from jax.experimental import pallas as pl
from jax.experimental.pallas import tpu as pltpu
from jax.experimental.pallas import tpu_sc as plsc

N_NODES = 10000
D = 128
N_REL = 12
N_BASES = 30
N_EDGES = 320000

NC = 2
NS = 16
L = 16
CHUNK = 80
N_NODES_PAD = 10112
PAD_KEYS = N_NODES_PAD * N_REL
N_EDGES_PAD = 322560
EDGES_PER_SC = N_EDGES_PAD // NC
CHUNKS_PER_SC = EDGES_PER_SC // CHUNK
CHUNKS_PER_TILE = CHUNKS_PER_SC // NS
NT3 = CHUNKS_PER_TILE // 3
KEY_SLICE = PAD_KEYS // NS
ROW_SLICE = N_NODES_PAD // NS
DRAIN_SIZES = [CHUNK] * (ROW_SLICE // CHUNK) + [ROW_SLICE % CHUNK]

_SIGMA = [0] * D
for _j in range(D // 32):
    for _m in range(16):
        _SIGMA[32 * _j + 2 * _m] = 32 * _j + _m
        _SIGMA[32 * _j + 2 * _m + 1] = 32 * _j + 16 + _m
NB = 5
BLK = N_NODES // NB


def _sc_mesh():
    return plsc.VectorSubcoreMesh(core_axis_name="c", subcore_axis_name="s")


def _sc_compiler_params():
    return dataclasses.replace(pltpu.CompilerParams(),
                               needs_layout_passes=False,
                               use_tc_tiling_on_sc=False)


def _sc_hist(idx3):
    @functools.partial(
        pl.kernel,
        mesh=_sc_mesh(),
        out_type=jax.ShapeDtypeStruct((NC * PAD_KEYS,), jnp.float32),
        scratch_types=[
            pltpu.VMEM_SHARED((PAD_KEYS,), jnp.float32),
            pltpu.VMEM((KEY_SLICE,), jnp.float32),
        ] + [pltpu.VMEM((2 * CHUNK,), jnp.int32)] * 3
          + [pltpu.VMEM((CHUNK,), jnp.int32)] * 3
          + [pltpu.VMEM((CHUNK,), jnp.float32)]
          + [pltpu.SemaphoreType.DMA] * 6,
    )
    def hist(idx_hbm, out_hbm, cnt_sp, zbuf_v, i0_v, i1_v, i2_v,
             w0_v, w1_v, w2_v, ones_v, si0, si1, si2, ss0, ss1, ss2):
        c = lax.axis_index("c")
        s = lax.axis_index("s")
        ibufs = [i0_v, i1_v, i2_v]
        wbufs = [w0_v, w1_v, w2_v]
        isems = [si0, si1, si2]
        ssems = [ss0, ss1, ss2]

        @pl.loop(0, KEY_SLICE // L)
        def _(i):
            zbuf_v[pl.ds(i * L, L)] = jnp.full((L,), 0.0, jnp.float32)

        pltpu.sync_copy(zbuf_v, cnt_sp.at[pl.ds(s * KEY_SLICE, KEY_SLICE)])
        for j in range(CHUNK // L):
            ones_v[pl.ds(j * L, L)] = jnp.full((L,), 1.0, jnp.float32)

        def load(k, sl):
            ci = c * CHUNKS_PER_SC + s + k * NS
            pltpu.async_copy(
                idx_hbm.at[pl.ds(ci * (3 * CHUNK) + CHUNK, 2 * CHUNK)],
                ibufs[sl], isems[sl])

        def consume(sl):
            pltpu.make_async_copy(
                idx_hbm.at[pl.ds(0, 2 * CHUNK)], ibufs[sl], isems[sl]).wait()
            for j in range(CHUNK // L):
                sl_ = pl.ds(j * L, L)
                wbufs[sl][sl_] = (ibufs[sl][pl.ds(j * L, L)] * N_REL
                                  + ibufs[sl][pl.ds(CHUNK + j * L, L)])
            pltpu.async_copy(ones_v, cnt_sp.at[wbufs[sl]], ssems[sl],
                             add=True)

        def wait_scat(sl):
            pltpu.make_async_copy(
                ones_v, cnt_sp.at[wbufs[sl]], ssems[sl]).wait()

        plsc.subcore_barrier()
        load(0, 0)
        load(1, 1)

        @pl.loop(0, NT3)
        def _(t):
            for p in range(3):
                kc = 3 * t + p

                @pl.when(kc < CHUNKS_PER_TILE)
                def _():
                    consume(p)

                prev = (p + 2) % 3

                @pl.when((kc >= 1) & (kc <= CHUNKS_PER_TILE))
                def _():
                    wait_scat(prev)

                @pl.when(kc + 2 < CHUNKS_PER_TILE)
                def _():
                    load(kc + 2, prev)

        wait_scat((CHUNKS_PER_TILE - 1) % 3)
        plsc.subcore_barrier()
        pltpu.sync_copy(cnt_sp.at[pl.ds(s * KEY_SLICE, KEY_SLICE)], zbuf_v)
        pltpu.sync_copy(
            zbuf_v,
            out_hbm.at[pl.ds(c * PAD_KEYS + s * KEY_SLICE, KEY_SLICE)])

    return hist(idx3)


def _tc_weights(comp, bases2):
    def body(comp_ref, bases_ref, out_ref):
        out_ref[...] = jnp.dot(comp_ref[...], bases_ref[...],
                               preferred_element_type=jnp.float32)

    return pl.pallas_call(
        body,
        out_shape=jax.ShapeDtypeStruct((N_REL, D * D), jnp.float32),
    )(comp, bases2)


def _tc_transform(x_bf, wcat_bf):
    def body(x_ref, w_ref, out_ref):
        out_ref[...] = jnp.dot(x_ref[...], w_ref[...],
                               preferred_element_type=jnp.float32
                               ).astype(jnp.bfloat16)

    return pl.pallas_call(
        body,
        grid=(NB,),
        in_specs=[
            pl.BlockSpec((BLK, D), lambda b: (b, 0)),
            pl.BlockSpec((D, N_REL * D), lambda b: (0, 0)),
        ],
        out_specs=pl.BlockSpec((BLK, N_REL * D), lambda b: (b, 0)),
        out_shape=jax.ShapeDtypeStruct((N_NODES, N_REL * D), jnp.bfloat16),
    )(x_bf, wcat_bf)


def _tc_inv(cnt_part):
    def body(c_ref, out_ref):
        total = c_ref[0:1, :] + c_ref[1:2, :]
        out_ref[...] = jnp.where(total > 0.0,
                                 1.0 / jnp.maximum(total, 1.0), 0.0)

    return pl.pallas_call(
        body,
        out_shape=jax.ShapeDtypeStruct((1, PAD_KEYS), jnp.float32),
    )(cnt_part)


def _sc_scatter(idx3, h_i32, inv1d):
    @functools.partial(
        pl.kernel,
        mesh=_sc_mesh(),
        compiler_params=_sc_compiler_params(),
        out_type=jax.ShapeDtypeStruct((NC, N_NODES_PAD, D), jnp.float32),
        scratch_types=[
            pltpu.VMEM_SHARED((N_NODES_PAD, D), jnp.float32),
            pltpu.VMEM((3 * CHUNK,), jnp.int32),
        ] + [pltpu.VMEM((CHUNK,), jnp.int32)] * 9
          + [pltpu.VMEM((CHUNK, D // 2), jnp.int32)] * 3
          + [pltpu.VMEM((CHUNK, D), jnp.float32)] * 3
          + [pltpu.VMEM((CHUNK,), jnp.float32)] * 3
          + [pltpu.SemaphoreType.DMA] * 9,
    )
    def scatter(idx_hbm, h_hbm, inv_hbm, out_hbm, acc_sp, ibuf_v,
                g0_v, g1_v, g2_v, k0_v, k1_v, k2_v, d0_v, d1_v, d2_v,
                b0_v, b1_v, b2_v, f0_v, f1_v, f2_v, w0_v, w1_v, w2_v,
                sr0, sr1, sr2, sw0, sw1, sw2, ss0, ss1, ss2):
        c = lax.axis_index("c")
        s = lax.axis_index("s")
        gbufs = [g0_v, g1_v, g2_v]
        kbufs = [k0_v, k1_v, k2_v]
        dbufs = [d0_v, d1_v, d2_v]
        bbufs = [b0_v, b1_v, b2_v]
        fbufs = [f0_v, f1_v, f2_v]
        wbufs = [w0_v, w1_v, w2_v]
        rsems = [sr0, sr1, sr2]
        wsems = [sw0, sw1, sw2]
        ssems = [ss0, ss1, ss2]

        @pl.loop(0, CHUNK)
        def _(i):
            for j in range(D // L):
                f0_v[i, pl.ds(j * L, L)] = jnp.full((L,), 0.0, jnp.float32)

        for k, sz in enumerate(DRAIN_SIZES):
            pltpu.sync_copy(
                f0_v.at[pl.ds(0, sz)],
                acc_sp.at[pl.ds(s * ROW_SLICE + k * CHUNK, sz)])
        plsc.subcore_barrier()

        def load(k, sl):
            ci = c * CHUNKS_PER_SC + s + k * NS
            pltpu.sync_copy(idx_hbm.at[pl.ds(ci * (3 * CHUNK), 3 * CHUNK)],
                            ibuf_v)
            gk, wk, dk = gbufs[sl], kbufs[sl], dbufs[sl]
            for j in range(CHUNK // L):
                sl_ = pl.ds(j * L, L)
                src_l = ibuf_v[pl.ds(j * L, L)]
                dst_l = ibuf_v[pl.ds(CHUNK + j * L, L)]
                et_l = ibuf_v[pl.ds(2 * CHUNK + j * L, L)]
                gk[sl_] = src_l * N_REL + et_l
                wk[sl_] = dst_l * N_REL + et_l
                dk[sl_] = dst_l
            pltpu.async_copy(h_hbm.at[gk], bbufs[sl], rsems[sl])
            pltpu.async_copy(inv_hbm.at[wk], wbufs[sl], wsems[sl])

        def consume(sl):
            rows_bf, rows_f, w = bbufs[sl], fbufs[sl], wbufs[sl]
            pltpu.make_async_copy(h_hbm.at[gbufs[sl]], rows_bf,
                                  rsems[sl]).wait()
            pltpu.make_async_copy(inv_hbm.at[kbufs[sl]], w,
                                  wsems[sl]).wait()
            himask = jnp.full((L,), -65536, jnp.int32)

            @pl.loop(0, CHUNK // L)
            def _(g):
                i0 = g * L
                wblk = w[pl.ds(i0, L)]
                for e in range(L):
                    we = wblk[e]
                    for j in range(D // 32):
                        w16 = rows_bf[i0 + e, pl.ds(L * j, L)]
                        lo = plsc.bitcast(
                            jax.lax.shift_left(w16, 16), jnp.float32)
                        hi = plsc.bitcast(w16 & himask, jnp.float32)
                        rows_f[i0 + e, pl.ds(32 * j, L)] = lo * we
                        rows_f[i0 + e, pl.ds(32 * j + L, L)] = hi * we

            pltpu.async_copy(rows_f, acc_sp.at[dbufs[sl]], ssems[sl],
                             add=True)

        def wait_scat(sl):
            pltpu.make_async_copy(fbufs[sl], acc_sp.at[dbufs[sl]],
                                  ssems[sl]).wait()

        load(0, 0)
        load(1, 1)

        @pl.loop(0, NT3)
        def _(t):
            for p in range(3):
                kc = 3 * t + p

                @pl.when(kc < CHUNKS_PER_TILE)
                def _():
                    consume(p)

                prev = (p + 2) % 3

                @pl.when((kc >= 1) & (kc <= CHUNKS_PER_TILE))
                def _():
                    wait_scat(prev)

                @pl.when(kc + 2 < CHUNKS_PER_TILE)
                def _():
                    load(kc + 2, prev)

        wait_scat((CHUNKS_PER_TILE - 1) % 3)
        plsc.subcore_barrier()
        for k, sz in enumerate(DRAIN_SIZES):
            off = s * ROW_SLICE + k * CHUNK
            pltpu.sync_copy(acc_sp.at[pl.ds(off, sz)], f0_v.at[pl.ds(0, sz)])
            pltpu.sync_copy(
                f0_v.at[pl.ds(0, sz)],
                out_hbm.at[c, pl.ds(pl.multiple_of(off, 8), sz)])

    return scatter(idx3, h_i32, inv1d)


def _tc_final(part, x, root, bias2d):
    def body(p_ref, x_ref, r_ref, b_ref, o_ref):
        acc = (p_ref[0] + p_ref[1]
               + jnp.dot(x_ref[...], r_ref[...],
                         preferred_element_type=jnp.float32)
               + b_ref[...])
        o_ref[...] = jnp.maximum(acc, 0.0)

    return pl.pallas_call(
        body,
        grid=(NB,),
        in_specs=[
            pl.BlockSpec((NC, BLK, D), lambda b: (0, b, 0)),
            pl.BlockSpec((BLK, D), lambda b: (b, 0)),
            pl.BlockSpec((D, D), lambda b: (0, 0)),
            pl.BlockSpec((1, D), lambda b: (0, 0)),
        ],
        out_specs=pl.BlockSpec((BLK, D), lambda b: (b, 0)),
        out_shape=jax.ShapeDtypeStruct((N_NODES, D), jnp.float32),
    )(part, x, root, bias2d)


def kernel(edge_type, edge_index, x, bases, comp, root, bias):
    et = edge_type.astype(jnp.int32)
    src = edge_index[0].astype(jnp.int32)
    dst = edge_index[1].astype(jnp.int32)

    n_pad = N_EDGES_PAD - N_EDGES
    pad_iota = jnp.arange(n_pad, dtype=jnp.int32)
    src_p = jnp.concatenate([src, pad_iota % N_NODES])
    dst_p = jnp.concatenate([dst, N_NODES + pad_iota % (N_NODES_PAD - N_NODES)])
    et_p = jnp.concatenate([et, pad_iota % N_REL])
    idx3 = jnp.stack(
        [src_p.reshape(-1, CHUNK), dst_p.reshape(-1, CHUNK),
         et_p.reshape(-1, CHUNK)], axis=1).reshape(-1)

    sigma = jnp.asarray(_SIGMA, dtype=jnp.int32)
    wall = _tc_weights(comp, bases[:, :, sigma].reshape(N_BASES, D * D))
    wcat = wall.reshape(N_REL, D, D).transpose(1, 0, 2).reshape(D, N_REL * D)
    h2 = _tc_transform(x.astype(jnp.bfloat16), wcat.astype(jnp.bfloat16))
    h_i32 = jax.lax.bitcast_convert_type(
        h2.reshape(N_NODES * N_REL, D // 2, 2), jnp.int32)
    cnt = _sc_hist(idx3).reshape(NC, PAD_KEYS)
    inv = _tc_inv(cnt).reshape(PAD_KEYS)
    part = _sc_scatter(idx3, h_i32, inv)
    return _tc_final(part, x, root, bias.reshape(1, D))


# speedup vs baseline: 22.0024x; 22.0024x over previous
import functools

import jax
import jax.numpy as jnp
from jax import lax
from jax.experimental import pallas as pl
from jax.experimental.pallas import tpu as pltpu
from jax.experimental.pallas import tpu_sc as plsc

N_NODES = 10000
D = 128
N_REL = 12
N_BASES = 30
N_EDGES = 320000

NC = 2
NS = 16
L = 16
CHUNK = 112
N_NODES_PAD = 10112
PAD_KEYS = N_NODES_PAD * N_REL
N_EDGES_PAD = 322560
EDGES_PER_SC = N_EDGES_PAD // NC
CHUNKS_PER_SC = EDGES_PER_SC // CHUNK
CHUNKS_PER_TILE = CHUNKS_PER_SC // NS
NT3 = CHUNKS_PER_TILE // 3
KEY_SLICE = PAD_KEYS // NS
ROW_SLICE = N_NODES_PAD // NS
DRAIN_SIZES = [CHUNK] * (ROW_SLICE // CHUNK) + [ROW_SLICE % CHUNK]

NB = 5
BLK = N_NODES // NB


def _sc_mesh():
    return plsc.VectorSubcoreMesh(core_axis_name="c", subcore_axis_name="s")


def _sc_hist(idx3):
    @functools.partial(
        pl.kernel,
        mesh=_sc_mesh(),
        out_type=jax.ShapeDtypeStruct((NC * PAD_KEYS,), jnp.float32),
        scratch_types=[
            pltpu.VMEM_SHARED((PAD_KEYS,), jnp.float32),
            pltpu.VMEM((KEY_SLICE,), jnp.float32),
        ] + [pltpu.VMEM((2 * CHUNK,), jnp.int32)] * 3
          + [pltpu.VMEM((CHUNK,), jnp.int32)] * 3
          + [pltpu.VMEM((CHUNK,), jnp.float32)]
          + [pltpu.SemaphoreType.DMA] * 6,
    )
    def hist(idx_hbm, out_hbm, cnt_sp, zbuf_v, i0_v, i1_v, i2_v,
             w0_v, w1_v, w2_v, ones_v, si0, si1, si2, ss0, ss1, ss2):
        c = lax.axis_index("c")
        s = lax.axis_index("s")
        ibufs = [i0_v, i1_v, i2_v]
        wbufs = [w0_v, w1_v, w2_v]
        isems = [si0, si1, si2]
        ssems = [ss0, ss1, ss2]

        @pl.loop(0, KEY_SLICE // L)
        def _(i):
            zbuf_v[pl.ds(i * L, L)] = jnp.full((L,), 0.0, jnp.float32)

        pltpu.sync_copy(zbuf_v, cnt_sp.at[pl.ds(s * KEY_SLICE, KEY_SLICE)])
        for j in range(CHUNK // L):
            ones_v[pl.ds(j * L, L)] = jnp.full((L,), 1.0, jnp.float32)

        def load(k, sl):
            ci = c * CHUNKS_PER_SC + s + k * NS
            pltpu.async_copy(
                idx_hbm.at[pl.ds(ci * (3 * CHUNK) + CHUNK, 2 * CHUNK)],
                ibufs[sl], isems[sl])

        def consume(sl):
            pltpu.make_async_copy(
                idx_hbm.at[pl.ds(0, 2 * CHUNK)], ibufs[sl], isems[sl]).wait()
            for j in range(CHUNK // L):
                sl_ = pl.ds(j * L, L)
                wbufs[sl][sl_] = (ibufs[sl][pl.ds(j * L, L)] * N_REL
                                  + ibufs[sl][pl.ds(CHUNK + j * L, L)])
            pltpu.async_copy(ones_v, cnt_sp.at[wbufs[sl]], ssems[sl],
                             add=True)

        def wait_scat(sl):
            pltpu.make_async_copy(
                ones_v, cnt_sp.at[wbufs[sl]], ssems[sl]).wait()

        plsc.subcore_barrier()
        load(0, 0)
        load(1, 1)

        @pl.loop(0, NT3)
        def _(t):
            for p in range(3):
                kc = 3 * t + p

                @pl.when(kc < CHUNKS_PER_TILE)
                def _():
                    consume(p)

                prev = (p + 2) % 3

                @pl.when((kc >= 1) & (kc <= CHUNKS_PER_TILE))
                def _():
                    wait_scat(prev)

                @pl.when(kc + 2 < CHUNKS_PER_TILE)
                def _():
                    load(kc + 2, prev)

        wait_scat((CHUNKS_PER_TILE - 1) % 3)
        plsc.subcore_barrier()
        pltpu.sync_copy(cnt_sp.at[pl.ds(s * KEY_SLICE, KEY_SLICE)], zbuf_v)
        pltpu.sync_copy(
            zbuf_v,
            out_hbm.at[pl.ds(c * PAD_KEYS + s * KEY_SLICE, KEY_SLICE)])

    return hist(idx3)


def _tc_weights(comp, bases2):
    def body(comp_ref, bases_ref, out_ref):
        out_ref[...] = jnp.dot(comp_ref[...], bases_ref[...],
                               preferred_element_type=jnp.float32)

    return pl.pallas_call(
        body,
        out_shape=jax.ShapeDtypeStruct((N_REL, D * D), jnp.float32),
    )(comp, bases2)


def _tc_transform(x, wcat):
    def body(x_ref, w_ref, out_ref):
        out_ref[...] = jnp.dot(x_ref[...], w_ref[...],
                               preferred_element_type=jnp.float32)

    return pl.pallas_call(
        body,
        grid=(NB,),
        in_specs=[
            pl.BlockSpec((BLK, D), lambda b: (b, 0)),
            pl.BlockSpec((D, N_REL * D), lambda b: (0, 0)),
        ],
        out_specs=pl.BlockSpec((BLK, N_REL * D), lambda b: (b, 0)),
        out_shape=jax.ShapeDtypeStruct((N_NODES, N_REL * D), jnp.float32),
    )(x, wcat)


def _tc_inv(cnt_part):
    def body(c_ref, out_ref):
        total = c_ref[0:1, :] + c_ref[1:2, :]
        out_ref[...] = jnp.where(total > 0.0,
                                 1.0 / jnp.maximum(total, 1.0), 0.0)

    return pl.pallas_call(
        body,
        out_shape=jax.ShapeDtypeStruct((1, PAD_KEYS), jnp.float32),
    )(cnt_part)


def _sc_scatter(idx3, h, inv1d):
    @functools.partial(
        pl.kernel,
        mesh=_sc_mesh(),
        out_type=jax.ShapeDtypeStruct((NC, N_NODES_PAD, D), jnp.float32),
        scratch_types=[
            pltpu.VMEM_SHARED((N_NODES_PAD, D), jnp.float32),
            pltpu.VMEM((3 * CHUNK,), jnp.int32),
        ] + [pltpu.VMEM((CHUNK,), jnp.int32)] * 9
          + [pltpu.VMEM((CHUNK, D), jnp.float32)] * 3
          + [pltpu.VMEM((CHUNK,), jnp.float32)] * 3
          + [pltpu.SemaphoreType.DMA] * 9,
    )
    def scatter(idx_hbm, h_hbm, inv_hbm, out_hbm, acc_sp, ibuf_v,
                g0_v, g1_v, g2_v, k0_v, k1_v, k2_v, d0_v, d1_v, d2_v,
                r0_v, r1_v, r2_v, w0_v, w1_v, w2_v,
                sr0, sr1, sr2, sw0, sw1, sw2, ss0, ss1, ss2):
        c = lax.axis_index("c")
        s = lax.axis_index("s")
        gbufs = [g0_v, g1_v, g2_v]
        kbufs = [k0_v, k1_v, k2_v]
        dbufs = [d0_v, d1_v, d2_v]
        rbufs = [r0_v, r1_v, r2_v]
        wbufs = [w0_v, w1_v, w2_v]
        rsems = [sr0, sr1, sr2]
        wsems = [sw0, sw1, sw2]
        ssems = [ss0, ss1, ss2]

        @pl.loop(0, CHUNK)
        def _(i):
            for j in range(D // L):
                r0_v[i, pl.ds(j * L, L)] = jnp.full((L,), 0.0, jnp.float32)

        for k, sz in enumerate(DRAIN_SIZES):
            pltpu.sync_copy(
                r0_v.at[pl.ds(0, sz)],
                acc_sp.at[pl.ds(s * ROW_SLICE + k * CHUNK, sz)])
        plsc.subcore_barrier()

        def load(k, sl):
            ci = c * CHUNKS_PER_SC + s + k * NS
            pltpu.sync_copy(idx_hbm.at[pl.ds(ci * (3 * CHUNK), 3 * CHUNK)],
                            ibuf_v)
            gk, wk, dk = gbufs[sl], kbufs[sl], dbufs[sl]
            for j in range(CHUNK // L):
                sl_ = pl.ds(j * L, L)
                src_l = ibuf_v[pl.ds(j * L, L)]
                dst_l = ibuf_v[pl.ds(CHUNK + j * L, L)]
                et_l = ibuf_v[pl.ds(2 * CHUNK + j * L, L)]
                gk[sl_] = src_l * N_REL + et_l
                wk[sl_] = dst_l * N_REL + et_l
                dk[sl_] = dst_l
            pltpu.async_copy(h_hbm.at[gk], rbufs[sl], rsems[sl])
            pltpu.async_copy(inv_hbm.at[wk], wbufs[sl], wsems[sl])

        def consume(sl):
            rows, w = rbufs[sl], wbufs[sl]
            pltpu.make_async_copy(h_hbm.at[gbufs[sl]], rows,
                                  rsems[sl]).wait()
            pltpu.make_async_copy(inv_hbm.at[kbufs[sl]], w,
                                  wsems[sl]).wait()

            @pl.loop(0, CHUNK // L)
            def _(g):
                i0 = g * L
                wblk = w[pl.ds(i0, L)]
                for e in range(L):
                    we = wblk[e]
                    for j in range(D // L):
                        sl2 = pl.ds(j * L, L)
                        rows[i0 + e, sl2] = rows[i0 + e, sl2] * we

            pltpu.async_copy(rows, acc_sp.at[dbufs[sl]], ssems[sl], add=True)

        def wait_scat(sl):
            pltpu.make_async_copy(rbufs[sl], acc_sp.at[dbufs[sl]],
                                  ssems[sl]).wait()

        load(0, 0)
        load(1, 1)

        @pl.loop(0, NT3)
        def _(t):
            for p in range(3):
                kc = 3 * t + p

                @pl.when(kc < CHUNKS_PER_TILE)
                def _():
                    consume(p)

                prev = (p + 2) % 3

                @pl.when((kc >= 1) & (kc <= CHUNKS_PER_TILE))
                def _():
                    wait_scat(prev)

                @pl.when(kc + 2 < CHUNKS_PER_TILE)
                def _():
                    load(kc + 2, prev)

        wait_scat((CHUNKS_PER_TILE - 1) % 3)
        plsc.subcore_barrier()
        for k, sz in enumerate(DRAIN_SIZES):
            off = s * ROW_SLICE + k * CHUNK
            pltpu.sync_copy(acc_sp.at[pl.ds(off, sz)], r0_v.at[pl.ds(0, sz)])
            pltpu.sync_copy(
                r0_v.at[pl.ds(0, sz)],
                out_hbm.at[c, pl.ds(pl.multiple_of(off, 8), sz)])

    return scatter(idx3, h, inv1d)


def _tc_final(part, x, root, bias2d):
    def body(p_ref, x_ref, r_ref, b_ref, o_ref):
        acc = (p_ref[0] + p_ref[1]
               + jnp.dot(x_ref[...], r_ref[...],
                         preferred_element_type=jnp.float32)
               + b_ref[...])
        o_ref[...] = jnp.maximum(acc, 0.0)

    return pl.pallas_call(
        body,
        grid=(NB,),
        in_specs=[
            pl.BlockSpec((NC, BLK, D), lambda b: (0, b, 0)),
            pl.BlockSpec((BLK, D), lambda b: (b, 0)),
            pl.BlockSpec((D, D), lambda b: (0, 0)),
            pl.BlockSpec((1, D), lambda b: (0, 0)),
        ],
        out_specs=pl.BlockSpec((BLK, D), lambda b: (b, 0)),
        out_shape=jax.ShapeDtypeStruct((N_NODES, D), jnp.float32),
    )(part, x, root, bias2d)


def kernel(edge_type, edge_index, x, bases, comp, root, bias):
    et = edge_type.astype(jnp.int32)
    src = edge_index[0].astype(jnp.int32)
    dst = edge_index[1].astype(jnp.int32)

    n_pad = N_EDGES_PAD - N_EDGES
    pad_iota = jnp.arange(n_pad, dtype=jnp.int32)
    src_p = jnp.concatenate([src, pad_iota % N_NODES])
    dst_p = jnp.concatenate([dst, N_NODES + pad_iota % (N_NODES_PAD - N_NODES)])
    et_p = jnp.concatenate([et, pad_iota % N_REL])
    idx3 = jnp.stack(
        [src_p.reshape(-1, CHUNK), dst_p.reshape(-1, CHUNK),
         et_p.reshape(-1, CHUNK)], axis=1).reshape(-1)

    wall = _tc_weights(comp, bases.reshape(N_BASES, D * D))
    wcat = wall.reshape(N_REL, D, D).transpose(1, 0, 2).reshape(D, N_REL * D)
    h = _tc_transform(x, wcat).reshape(N_NODES * N_REL, D)
    cnt = _sc_hist(idx3).reshape(NC, PAD_KEYS)
    inv = _tc_inv(cnt).reshape(PAD_KEYS)
    part = _sc_scatter(idx3, h, inv)
    return _tc_final(part, x, root, bias.reshape(1, D))


# speedup vs baseline: 24.7339x; 1.1241x over previous
import functools

import jax
import jax.numpy as jnp
from jax import lax
from jax.experimental import pallas as pl
from jax.experimental.pallas import tpu as pltpu
from jax.experimental.pallas import tpu_sc as plsc

N_NODES = 10000
D = 128
N_REL = 12
N_BASES = 30
N_EDGES = 320000

NC = 2
NS = 16
L = 16
CHUNK = 112
N_NODES_PAD = 10112
PAD_KEYS = N_NODES_PAD * N_REL
N_EDGES_PAD = 322560
EDGES_PER_SC = N_EDGES_PAD // NC
CHUNKS_PER_SC = EDGES_PER_SC // CHUNK
CHUNKS_PER_TILE = CHUNKS_PER_SC // NS
NT3 = CHUNKS_PER_TILE // 3
KEY_SLICE = PAD_KEYS // NS
ROW_SLICE = N_NODES_PAD // NS
DRAIN_SIZES = [CHUNK] * (ROW_SLICE // CHUNK) + [ROW_SLICE % CHUNK]

NB = 5
BLK = N_NODES // NB


def _sc_mesh():
    return plsc.VectorSubcoreMesh(core_axis_name="c", subcore_axis_name="s")


def _sc_hist(idx3):
    @functools.partial(
        pl.kernel,
        mesh=_sc_mesh(),
        out_type=jax.ShapeDtypeStruct((NC * PAD_KEYS,), jnp.float32),
        scratch_types=[
            pltpu.VMEM_SHARED((PAD_KEYS,), jnp.float32),
            pltpu.VMEM((KEY_SLICE,), jnp.float32),
        ] + [pltpu.VMEM((2 * CHUNK,), jnp.int32)] * 3
          + [pltpu.VMEM((CHUNK,), jnp.int32)] * 3
          + [pltpu.VMEM((CHUNK,), jnp.float32)]
          + [pltpu.SemaphoreType.DMA] * 6,
    )
    def hist(idx_hbm, out_hbm, cnt_sp, zbuf_v, i0_v, i1_v, i2_v,
             w0_v, w1_v, w2_v, ones_v, si0, si1, si2, ss0, ss1, ss2):
        c = lax.axis_index("c")
        s = lax.axis_index("s")
        ibufs = [i0_v, i1_v, i2_v]
        wbufs = [w0_v, w1_v, w2_v]
        isems = [si0, si1, si2]
        ssems = [ss0, ss1, ss2]

        @pl.loop(0, KEY_SLICE // L)
        def _(i):
            zbuf_v[pl.ds(i * L, L)] = jnp.full((L,), 0.0, jnp.float32)

        pltpu.sync_copy(zbuf_v, cnt_sp.at[pl.ds(s * KEY_SLICE, KEY_SLICE)])
        for j in range(CHUNK // L):
            ones_v[pl.ds(j * L, L)] = jnp.full((L,), 1.0, jnp.float32)

        def load(k, sl):
            ci = c * CHUNKS_PER_SC + s + k * NS
            pltpu.async_copy(
                idx_hbm.at[pl.ds(ci * (3 * CHUNK) + CHUNK, 2 * CHUNK)],
                ibufs[sl], isems[sl])

        def consume(sl):
            pltpu.make_async_copy(
                idx_hbm.at[pl.ds(0, 2 * CHUNK)], ibufs[sl], isems[sl]).wait()
            for j in range(CHUNK // L):
                sl_ = pl.ds(j * L, L)
                wbufs[sl][sl_] = (ibufs[sl][pl.ds(j * L, L)] * N_REL
                                  + ibufs[sl][pl.ds(CHUNK + j * L, L)])
            pltpu.async_copy(ones_v, cnt_sp.at[wbufs[sl]], ssems[sl],
                             add=True)

        def wait_scat(sl):
            pltpu.make_async_copy(
                ones_v, cnt_sp.at[wbufs[sl]], ssems[sl]).wait()

        plsc.subcore_barrier()
        load(0, 0)
        load(1, 1)

        @pl.loop(0, NT3)
        def _(t):
            for p in range(3):
                kc = 3 * t + p

                @pl.when(kc < CHUNKS_PER_TILE)
                def _():
                    consume(p)

                prev = (p + 2) % 3

                @pl.when((kc >= 1) & (kc <= CHUNKS_PER_TILE))
                def _():
                    wait_scat(prev)

                @pl.when(kc + 2 < CHUNKS_PER_TILE)
                def _():
                    load(kc + 2, prev)

        wait_scat((CHUNKS_PER_TILE - 1) % 3)
        plsc.subcore_barrier()
        pltpu.sync_copy(cnt_sp.at[pl.ds(s * KEY_SLICE, KEY_SLICE)], zbuf_v)
        pltpu.sync_copy(
            zbuf_v,
            out_hbm.at[pl.ds(c * PAD_KEYS + s * KEY_SLICE, KEY_SLICE)])

    return hist(idx3)


def _tc_weights(comp, bases2):
    def body(comp_ref, bases_ref, out_ref):
        out_ref[...] = jnp.dot(comp_ref[...], bases_ref[...],
                               preferred_element_type=jnp.float32)

    return pl.pallas_call(
        body,
        out_shape=jax.ShapeDtypeStruct((N_REL, D * D), jnp.float32),
    )(comp, bases2)


def _tc_transform(x, wall):
    def body(x_ref, w_ref, out_ref):
        out_ref[...] = jnp.dot(x_ref[...], w_ref[0],
                               preferred_element_type=jnp.float32)

    return pl.pallas_call(
        body,
        grid=(NB, N_REL),
        in_specs=[
            pl.BlockSpec((BLK, D), lambda b, r: (b, 0)),
            pl.BlockSpec((1, D, D), lambda b, r: (r, 0, 0)),
        ],
        out_specs=pl.BlockSpec((BLK, D), lambda b, r: (r * NB + b, 0)),
        out_shape=jax.ShapeDtypeStruct((N_REL * N_NODES, D), jnp.float32),
    )(x, wall)


def _tc_inv(cnt_part):
    def body(c_ref, out_ref):
        total = c_ref[0:1, :] + c_ref[1:2, :]
        out_ref[...] = jnp.where(total > 0.0,
                                 1.0 / jnp.maximum(total, 1.0), 0.0)

    return pl.pallas_call(
        body,
        out_shape=jax.ShapeDtypeStruct((1, PAD_KEYS), jnp.float32),
    )(cnt_part)


def _sc_scatter(idx3, h, inv1d):
    @functools.partial(
        pl.kernel,
        mesh=_sc_mesh(),
        out_type=jax.ShapeDtypeStruct((NC, N_NODES_PAD, D), jnp.float32),
        scratch_types=[
            pltpu.VMEM_SHARED((N_NODES_PAD, D), jnp.float32),
            pltpu.VMEM((3 * CHUNK,), jnp.int32),
        ] + [pltpu.VMEM((CHUNK,), jnp.int32)] * 9
          + [pltpu.VMEM((CHUNK, D), jnp.float32)] * 3
          + [pltpu.VMEM((CHUNK,), jnp.float32)] * 3
          + [pltpu.SemaphoreType.DMA] * 9,
    )
    def scatter(idx_hbm, h_hbm, inv_hbm, out_hbm, acc_sp, ibuf_v,
                g0_v, g1_v, g2_v, k0_v, k1_v, k2_v, d0_v, d1_v, d2_v,
                r0_v, r1_v, r2_v, w0_v, w1_v, w2_v,
                sr0, sr1, sr2, sw0, sw1, sw2, ss0, ss1, ss2):
        c = lax.axis_index("c")
        s = lax.axis_index("s")
        gbufs = [g0_v, g1_v, g2_v]
        kbufs = [k0_v, k1_v, k2_v]
        dbufs = [d0_v, d1_v, d2_v]
        rbufs = [r0_v, r1_v, r2_v]
        wbufs = [w0_v, w1_v, w2_v]
        rsems = [sr0, sr1, sr2]
        wsems = [sw0, sw1, sw2]
        ssems = [ss0, ss1, ss2]

        @pl.loop(0, CHUNK)
        def _(i):
            for j in range(D // L):
                r0_v[i, pl.ds(j * L, L)] = jnp.full((L,), 0.0, jnp.float32)

        for k, sz in enumerate(DRAIN_SIZES):
            pltpu.sync_copy(
                r0_v.at[pl.ds(0, sz)],
                acc_sp.at[pl.ds(s * ROW_SLICE + k * CHUNK, sz)])
        plsc.subcore_barrier()

        def load(k, sl):
            ci = c * CHUNKS_PER_SC + s + k * NS
            pltpu.sync_copy(idx_hbm.at[pl.ds(ci * (3 * CHUNK), 3 * CHUNK)],
                            ibuf_v)
            gk, wk, dk = gbufs[sl], kbufs[sl], dbufs[sl]
            for j in range(CHUNK // L):
                sl_ = pl.ds(j * L, L)
                src_l = ibuf_v[pl.ds(j * L, L)]
                dst_l = ibuf_v[pl.ds(CHUNK + j * L, L)]
                et_l = ibuf_v[pl.ds(2 * CHUNK + j * L, L)]
                gk[sl_] = et_l * N_NODES + src_l
                wk[sl_] = dst_l * N_REL + et_l
                dk[sl_] = dst_l
            pltpu.async_copy(h_hbm.at[gk], rbufs[sl], rsems[sl])
            pltpu.async_copy(inv_hbm.at[wk], wbufs[sl], wsems[sl])

        def consume(sl):
            rows, w = rbufs[sl], wbufs[sl]
            pltpu.make_async_copy(h_hbm.at[gbufs[sl]], rows,
                                  rsems[sl]).wait()
            pltpu.make_async_copy(inv_hbm.at[kbufs[sl]], w,
                                  wsems[sl]).wait()

            @pl.loop(0, CHUNK // L)
            def _(g):
                i0 = g * L
                wblk = w[pl.ds(i0, L)]
                for e in range(L):
                    we = wblk[e]
                    for j in range(D // L):
                        sl2 = pl.ds(j * L, L)
                        rows[i0 + e, sl2] = rows[i0 + e, sl2] * we

            pltpu.async_copy(rows, acc_sp.at[dbufs[sl]], ssems[sl], add=True)

        def wait_scat(sl):
            pltpu.make_async_copy(rbufs[sl], acc_sp.at[dbufs[sl]],
                                  ssems[sl]).wait()

        load(0, 0)
        load(1, 1)

        @pl.loop(0, NT3)
        def _(t):
            for p in range(3):
                kc = 3 * t + p

                @pl.when(kc < CHUNKS_PER_TILE)
                def _():
                    consume(p)

                prev = (p + 2) % 3

                @pl.when((kc >= 1) & (kc <= CHUNKS_PER_TILE))
                def _():
                    wait_scat(prev)

                @pl.when(kc + 2 < CHUNKS_PER_TILE)
                def _():
                    load(kc + 2, prev)

        wait_scat((CHUNKS_PER_TILE - 1) % 3)
        plsc.subcore_barrier()
        for k, sz in enumerate(DRAIN_SIZES):
            off = s * ROW_SLICE + k * CHUNK
            pltpu.sync_copy(acc_sp.at[pl.ds(off, sz)], r0_v.at[pl.ds(0, sz)])
            pltpu.sync_copy(
                r0_v.at[pl.ds(0, sz)],
                out_hbm.at[c, pl.ds(pl.multiple_of(off, 8), sz)])

    return scatter(idx3, h, inv1d)


def _tc_final(part, x, root, bias2d):
    def body(p_ref, x_ref, r_ref, b_ref, o_ref):
        acc = (p_ref[0] + p_ref[1]
               + jnp.dot(x_ref[...], r_ref[...],
                         preferred_element_type=jnp.float32)
               + b_ref[...])
        o_ref[...] = jnp.maximum(acc, 0.0)

    return pl.pallas_call(
        body,
        grid=(NB,),
        in_specs=[
            pl.BlockSpec((NC, BLK, D), lambda b: (0, b, 0)),
            pl.BlockSpec((BLK, D), lambda b: (b, 0)),
            pl.BlockSpec((D, D), lambda b: (0, 0)),
            pl.BlockSpec((1, D), lambda b: (0, 0)),
        ],
        out_specs=pl.BlockSpec((BLK, D), lambda b: (b, 0)),
        out_shape=jax.ShapeDtypeStruct((N_NODES, D), jnp.float32),
    )(part, x, root, bias2d)


def kernel(edge_type, edge_index, x, bases, comp, root, bias):
    et = edge_type.astype(jnp.int32)
    src = edge_index[0].astype(jnp.int32)
    dst = edge_index[1].astype(jnp.int32)

    n_pad = N_EDGES_PAD - N_EDGES
    pad_iota = jnp.arange(n_pad, dtype=jnp.int32)
    src_p = jnp.concatenate([src, pad_iota % N_NODES])
    dst_p = jnp.concatenate([dst, N_NODES + pad_iota % (N_NODES_PAD - N_NODES)])
    et_p = jnp.concatenate([et, pad_iota % N_REL])
    idx3 = jnp.stack(
        [src_p.reshape(-1, CHUNK), dst_p.reshape(-1, CHUNK),
         et_p.reshape(-1, CHUNK)], axis=1).reshape(-1)

    wall = _tc_weights(comp, bases.reshape(N_BASES, D * D))
    h = _tc_transform(x, wall.reshape(N_REL, D, D))
    cnt = _sc_hist(idx3).reshape(NC, PAD_KEYS)
    inv = _tc_inv(cnt).reshape(PAD_KEYS)
    part = _sc_scatter(idx3, h, inv)
    return _tc_final(part, x, root, bias.reshape(1, D))


# speedup vs baseline: 26.0344x; 1.0526x over previous
import functools

import jax
import jax.numpy as jnp
from jax import lax
from jax.experimental import pallas as pl
from jax.experimental.pallas import tpu as pltpu
from jax.experimental.pallas import tpu_sc as plsc

N_NODES = 10000
D = 128
N_REL = 12
N_BASES = 30
N_EDGES = 320000

NC = 2
NS = 16
L = 16
CHUNK = 112
N_NODES_PAD = 10112
PAD_KEYS = N_NODES_PAD * N_REL
N_EDGES_PAD = 322560
EDGES_PER_SC = N_EDGES_PAD // NC
CHUNKS_PER_SC = EDGES_PER_SC // CHUNK
CHUNKS_PER_TILE = CHUNKS_PER_SC // NS
NT3 = CHUNKS_PER_TILE // 3
KEY_SLICE = PAD_KEYS // NS
ROW_SLICE = N_NODES_PAD // NS
DRAIN_SIZES = [CHUNK] * (ROW_SLICE // CHUNK) + [ROW_SLICE % CHUNK]

NB = 5
BLK = N_NODES // NB


def _sc_mesh():
    return plsc.VectorSubcoreMesh(core_axis_name="c", subcore_axis_name="s")


def _sc_hist(idx3):
    @functools.partial(
        pl.kernel,
        mesh=_sc_mesh(),
        out_type=jax.ShapeDtypeStruct((NC * PAD_KEYS,), jnp.float32),
        scratch_types=[
            pltpu.VMEM_SHARED((PAD_KEYS,), jnp.float32),
            pltpu.VMEM((KEY_SLICE,), jnp.float32),
        ] + [pltpu.VMEM((2 * CHUNK,), jnp.int32)] * 3
          + [pltpu.VMEM((CHUNK,), jnp.int32)] * 3
          + [pltpu.VMEM((CHUNK,), jnp.float32)]
          + [pltpu.SemaphoreType.DMA] * 6,
    )
    def hist(idx_hbm, out_hbm, cnt_sp, zbuf_v, i0_v, i1_v, i2_v,
             w0_v, w1_v, w2_v, ones_v, si0, si1, si2, ss0, ss1, ss2):
        c = lax.axis_index("c")
        s = lax.axis_index("s")
        ibufs = [i0_v, i1_v, i2_v]
        wbufs = [w0_v, w1_v, w2_v]
        isems = [si0, si1, si2]
        ssems = [ss0, ss1, ss2]

        @pl.loop(0, KEY_SLICE // L)
        def _(i):
            zbuf_v[pl.ds(i * L, L)] = jnp.full((L,), 0.0, jnp.float32)

        pltpu.sync_copy(zbuf_v, cnt_sp.at[pl.ds(s * KEY_SLICE, KEY_SLICE)])
        for j in range(CHUNK // L):
            ones_v[pl.ds(j * L, L)] = jnp.full((L,), 1.0, jnp.float32)

        def load(k, sl):
            ci = c * CHUNKS_PER_SC + s + k * NS
            pltpu.async_copy(
                idx_hbm.at[pl.ds(ci * (3 * CHUNK) + CHUNK, 2 * CHUNK)],
                ibufs[sl], isems[sl])

        def consume(sl):
            pltpu.make_async_copy(
                idx_hbm.at[pl.ds(0, 2 * CHUNK)], ibufs[sl], isems[sl]).wait()
            for j in range(CHUNK // L):
                sl_ = pl.ds(j * L, L)
                wbufs[sl][sl_] = (ibufs[sl][pl.ds(j * L, L)] * N_REL
                                  + ibufs[sl][pl.ds(CHUNK + j * L, L)])
            pltpu.async_copy(ones_v, cnt_sp.at[wbufs[sl]], ssems[sl],
                             add=True)

        def wait_scat(sl):
            pltpu.make_async_copy(
                ones_v, cnt_sp.at[wbufs[sl]], ssems[sl]).wait()

        plsc.subcore_barrier()
        load(0, 0)
        load(1, 1)

        @pl.loop(0, NT3)
        def _(t):
            for p in range(3):
                kc = 3 * t + p

                @pl.when(kc < CHUNKS_PER_TILE)
                def _():
                    consume(p)

                prev = (p + 2) % 3

                @pl.when((kc >= 1) & (kc <= CHUNKS_PER_TILE))
                def _():
                    wait_scat(prev)

                @pl.when(kc + 2 < CHUNKS_PER_TILE)
                def _():
                    load(kc + 2, prev)

        wait_scat((CHUNKS_PER_TILE - 1) % 3)
        plsc.subcore_barrier()
        pltpu.sync_copy(cnt_sp.at[pl.ds(s * KEY_SLICE, KEY_SLICE)], zbuf_v)
        pltpu.sync_copy(
            zbuf_v,
            out_hbm.at[pl.ds(c * PAD_KEYS + s * KEY_SLICE, KEY_SLICE)])

    return hist(idx3)


def _tc_weights(comp, bases2):
    def body(comp_ref, bases_ref, out_ref):
        out_ref[...] = jnp.dot(comp_ref[...], bases_ref[...],
                               preferred_element_type=jnp.float32)

    return pl.pallas_call(
        body,
        out_shape=jax.ShapeDtypeStruct((N_REL, D * D), jnp.float32),
    )(comp, bases2)


def _tc_transform(x, wall):
    def body(x_ref, w_ref, out_ref):
        out_ref[...] = jnp.dot(x_ref[...], w_ref[0],
                               preferred_element_type=jnp.float32)

    return pl.pallas_call(
        body,
        grid=(NB, N_REL),
        in_specs=[
            pl.BlockSpec((BLK, D), lambda b, r: (b, 0)),
            pl.BlockSpec((1, D, D), lambda b, r: (r, 0, 0)),
        ],
        out_specs=pl.BlockSpec((BLK, D), lambda b, r: (r * NB + b, 0)),
        out_shape=jax.ShapeDtypeStruct((N_REL * N_NODES, D), jnp.float32),
    )(x, wall)


def _tc_inv(cnt_part):
    def body(c_ref, out_ref):
        total = c_ref[0:1, :] + c_ref[1:2, :]
        out_ref[...] = jnp.where(total > 0.0,
                                 1.0 / jnp.maximum(total, 1.0), 0.0)

    return pl.pallas_call(
        body,
        out_shape=jax.ShapeDtypeStruct((1, PAD_KEYS), jnp.float32),
    )(cnt_part)


def _sc_scatter(idx3, h, inv1d):
    @functools.partial(
        pl.kernel,
        mesh=_sc_mesh(),
        out_type=jax.ShapeDtypeStruct((NC, N_NODES_PAD, D), jnp.float32),
        scratch_types=[
            pltpu.VMEM_SHARED((N_NODES_PAD, D), jnp.float32),
        ] + [pltpu.VMEM((3 * CHUNK,), jnp.int32)] * 3
          + [pltpu.VMEM((CHUNK,), jnp.int32)] * 9
          + [pltpu.VMEM((CHUNK, D), jnp.float32)] * 3
          + [pltpu.VMEM((CHUNK,), jnp.float32)] * 3
          + [pltpu.SemaphoreType.DMA] * 12,
    )
    def scatter(idx_hbm, h_hbm, inv_hbm, out_hbm, acc_sp, ia_v, ib_v, ic_v,
                g0_v, g1_v, g2_v, k0_v, k1_v, k2_v, d0_v, d1_v, d2_v,
                r0_v, r1_v, r2_v, w0_v, w1_v, w2_v,
                sr0, sr1, sr2, sw0, sw1, sw2, ss0, ss1, ss2, si0, si1, si2):
        c = lax.axis_index("c")
        s = lax.axis_index("s")
        ibufs = [ia_v, ib_v, ic_v]
        isems = [si0, si1, si2]
        gbufs = [g0_v, g1_v, g2_v]
        kbufs = [k0_v, k1_v, k2_v]
        dbufs = [d0_v, d1_v, d2_v]
        rbufs = [r0_v, r1_v, r2_v]
        wbufs = [w0_v, w1_v, w2_v]
        rsems = [sr0, sr1, sr2]
        wsems = [sw0, sw1, sw2]
        ssems = [ss0, ss1, ss2]

        @pl.loop(0, CHUNK)
        def _(i):
            for j in range(D // L):
                r0_v[i, pl.ds(j * L, L)] = jnp.full((L,), 0.0, jnp.float32)

        for k, sz in enumerate(DRAIN_SIZES):
            pltpu.sync_copy(
                r0_v.at[pl.ds(0, sz)],
                acc_sp.at[pl.ds(s * ROW_SLICE + k * CHUNK, sz)])
        plsc.subcore_barrier()

        def issue_idx(k, b):
            ci = c * CHUNKS_PER_SC + s + k * NS
            pltpu.async_copy(idx_hbm.at[pl.ds(ci * (3 * CHUNK), 3 * CHUNK)],
                             ibufs[b], isems[b])

        def prep(k, sl):
            ib = ibufs[sl]
            pltpu.make_async_copy(idx_hbm.at[pl.ds(0, 3 * CHUNK)], ib,
                                  isems[sl]).wait()
            gk, wk, dk = gbufs[sl], kbufs[sl], dbufs[sl]
            for j in range(CHUNK // L):
                sl_ = pl.ds(j * L, L)
                src_l = ib[pl.ds(j * L, L)]
                dst_l = ib[pl.ds(CHUNK + j * L, L)]
                et_l = ib[pl.ds(2 * CHUNK + j * L, L)]
                gk[sl_] = et_l * N_NODES + src_l
                wk[sl_] = dst_l * N_REL + et_l
                dk[sl_] = dst_l
            pltpu.async_copy(h_hbm.at[gk], rbufs[sl], rsems[sl])
            pltpu.async_copy(inv_hbm.at[wk], wbufs[sl], wsems[sl])

        def consume(sl):
            rows, w = rbufs[sl], wbufs[sl]
            pltpu.make_async_copy(h_hbm.at[gbufs[sl]], rows,
                                  rsems[sl]).wait()
            pltpu.make_async_copy(inv_hbm.at[kbufs[sl]], w,
                                  wsems[sl]).wait()

            @pl.loop(0, CHUNK // L)
            def _(g):
                i0 = g * L
                wblk = w[pl.ds(i0, L)]
                for e in range(L):
                    we = wblk[e]
                    for j in range(D // L):
                        sl2 = pl.ds(j * L, L)
                        rows[i0 + e, sl2] = rows[i0 + e, sl2] * we

            pltpu.async_copy(rows, acc_sp.at[dbufs[sl]], ssems[sl], add=True)

        def wait_scat(sl):
            pltpu.make_async_copy(rbufs[sl], acc_sp.at[dbufs[sl]],
                                  ssems[sl]).wait()

        issue_idx(0, 0)
        issue_idx(1, 1)
        prep(0, 0)
        issue_idx(2, 2)
        prep(1, 1)
        issue_idx(3, 0)

        @pl.loop(0, NT3)
        def _(t):
            for p in range(3):
                kc = 3 * t + p

                @pl.when(kc < CHUNKS_PER_TILE)
                def _():
                    consume(p)

                prev = (p + 2) % 3

                @pl.when((kc >= 1) & (kc <= CHUNKS_PER_TILE))
                def _():
                    wait_scat(prev)

                @pl.when(kc + 2 < CHUNKS_PER_TILE)
                def _():
                    prep(kc + 2, prev)

                @pl.when(kc + 4 < CHUNKS_PER_TILE)
                def _():
                    issue_idx(kc + 4, (p + 1) % 3)

        wait_scat((CHUNKS_PER_TILE - 1) % 3)
        plsc.subcore_barrier()
        for k, sz in enumerate(DRAIN_SIZES):
            off = s * ROW_SLICE + k * CHUNK
            pltpu.sync_copy(acc_sp.at[pl.ds(off, sz)], r0_v.at[pl.ds(0, sz)])
            pltpu.sync_copy(
                r0_v.at[pl.ds(0, sz)],
                out_hbm.at[c, pl.ds(pl.multiple_of(off, 8), sz)])

    return scatter(idx3, h, inv1d)


def _tc_final(part, x, root, bias2d):
    def body(p_ref, x_ref, r_ref, b_ref, o_ref):
        acc = (p_ref[0] + p_ref[1]
               + jnp.dot(x_ref[...], r_ref[...],
                         preferred_element_type=jnp.float32)
               + b_ref[...])
        o_ref[...] = jnp.maximum(acc, 0.0)

    return pl.pallas_call(
        body,
        grid=(NB,),
        in_specs=[
            pl.BlockSpec((NC, BLK, D), lambda b: (0, b, 0)),
            pl.BlockSpec((BLK, D), lambda b: (b, 0)),
            pl.BlockSpec((D, D), lambda b: (0, 0)),
            pl.BlockSpec((1, D), lambda b: (0, 0)),
        ],
        out_specs=pl.BlockSpec((BLK, D), lambda b: (b, 0)),
        out_shape=jax.ShapeDtypeStruct((N_NODES, D), jnp.float32),
    )(part, x, root, bias2d)


def kernel(edge_type, edge_index, x, bases, comp, root, bias):
    et = edge_type.astype(jnp.int32)
    src = edge_index[0].astype(jnp.int32)
    dst = edge_index[1].astype(jnp.int32)

    n_pad = N_EDGES_PAD - N_EDGES
    pad_iota = jnp.arange(n_pad, dtype=jnp.int32)
    src_p = jnp.concatenate([src, pad_iota % N_NODES])
    dst_p = jnp.concatenate([dst, N_NODES + pad_iota % (N_NODES_PAD - N_NODES)])
    et_p = jnp.concatenate([et, pad_iota % N_REL])
    idx3 = jnp.stack(
        [src_p.reshape(-1, CHUNK), dst_p.reshape(-1, CHUNK),
         et_p.reshape(-1, CHUNK)], axis=1).reshape(-1)

    wall = _tc_weights(comp, bases.reshape(N_BASES, D * D))
    h = _tc_transform(x, wall.reshape(N_REL, D, D))
    cnt = _sc_hist(idx3).reshape(NC, PAD_KEYS)
    inv = _tc_inv(cnt).reshape(PAD_KEYS)
    part = _sc_scatter(idx3, h, inv)
    return _tc_final(part, x, root, bias.reshape(1, D))
